# baseline (speedup 1.0000x reference)
import functools

import jax
import jax.numpy as jnp
from jax.experimental import pallas as pl
from jax.experimental.pallas import tpu as pltpu

_NEG_SLOPE = 0.1
_BN_EPS = 1e-5
_VMEM_LIMIT = 60 * 1024 * 1024


def _leaky(v):
    return jnp.maximum(v, _NEG_SLOPE * v)


def _csp_kernel(x_ref, wh_ref, bh_ref, w1s_ref, b1s_ref, w2s_ref, b2s_ref,
                wz_ref, bz_ref, w4_ref, b4_ref,
                o_ref, *c3_refs, H, W, n_blocks, n_img):
    HW = H * W
    c_ = wh_ref.shape[1]
    col = jax.lax.broadcasted_iota(jnp.int32, (HW, 1), 0) % W
    m_left = (col != 0).astype(jnp.bfloat16)
    m_right = (col != (W - 1)).astype(jnp.bfloat16)

    scratches = c3_refs
    for img in range(n_img):
        c3_ref = scratches[img]
        base = img * HW
        xb = x_ref[base:base + HW, :].astype(jnp.bfloat16)

        y = _leaky((jnp.dot(xb, wh_ref[...],
                            preferred_element_type=jnp.float32)
                    + bh_ref[...]).astype(jnp.bfloat16))

        c3_ref[0:W + 1, :] = jnp.zeros((W + 1, 3 * c_), jnp.bfloat16)
        c3_ref[W + HW - 1:, :] = jnp.zeros((W + 1, 3 * c_), jnp.bfloat16)

        for blk in range(n_blocks):
            tb = _leaky((jnp.dot(y, w1s_ref[blk],
                                 preferred_element_type=jnp.float32)
                         + b1s_ref[blk]).astype(jnp.bfloat16))
            tl = tb * m_right
            tr = tb * m_left
            c3_ref[W + 1:W + 1 + HW, 0:c_] = tl
            c3_ref[W:W + HW, c_:2 * c_] = tb
            c3_ref[W - 1:W - 1 + HW, 2 * c_:3 * c_] = tr
            acc = jnp.dot(c3_ref[0:HW, :], w2s_ref[3 * blk],
                          preferred_element_type=jnp.float32)
            acc = acc + jnp.dot(c3_ref[W:W + HW, :], w2s_ref[3 * blk + 1],
                                preferred_element_type=jnp.float32)
            acc = acc + jnp.dot(c3_ref[2 * W:2 * W + HW, :],
                                w2s_ref[3 * blk + 2],
                                preferred_element_type=jnp.float32)
            y = _leaky((acc + b2s_ref[blk]).astype(jnp.bfloat16)) + y

        zin = jnp.concatenate([y, xb], axis=1)
        u = _leaky((jnp.dot(zin, wz_ref[...],
                            preferred_element_type=jnp.float32)
                    + bz_ref[...]).astype(jnp.bfloat16))
        v = (jnp.dot(u, w4_ref[...], preferred_element_type=jnp.float32)
             + b4_ref[...])
        o_ref[base:base + HW, :] = _leaky(v)


def _w1x1(w):
    return jnp.transpose(w[:, :, 0, 0], (1, 0))


def _fold_scale(gamma, var):
    return gamma * jax.lax.rsqrt(var + _BN_EPS)


def kernel(x, cv1_conv_w, cv1_conv_b, cv1_bn_gamma, cv1_bn_beta, cv1_bn_mean, cv1_bn_var, cv2_w, cv3_w, cv4_conv_w, cv4_conv_b, cv4_bn_gamma, cv4_bn_beta, cv4_bn_mean, cv4_bn_var, bn_gamma, bn_beta, bn_mean, bn_var, m0_cv1_conv_w, m0_cv1_conv_b, m0_cv1_bn_gamma, m0_cv1_bn_beta, m0_cv1_bn_mean, m0_cv1_bn_var, m0_cv2_conv_w, m0_cv2_conv_b, m0_cv2_bn_gamma, m0_cv2_bn_beta, m0_cv2_bn_mean, m0_cv2_bn_var, m1_cv1_conv_w, m1_cv1_conv_b, m1_cv1_bn_gamma, m1_cv1_bn_beta, m1_cv1_bn_mean, m1_cv1_bn_var, m1_cv2_conv_w, m1_cv2_conv_b, m1_cv2_bn_gamma, m1_cv2_bn_beta, m1_cv2_bn_mean, m1_cv2_bn_var, m2_cv1_conv_w, m2_cv1_conv_b, m2_cv1_bn_gamma, m2_cv1_bn_beta, m2_cv1_bn_mean, m2_cv1_bn_var, m2_cv2_conv_w, m2_cv2_conv_b, m2_cv2_bn_gamma, m2_cv2_bn_beta, m2_cv2_bn_mean, m2_cv2_bn_var):
    Nb, c1, H, W = x.shape
    HW = H * W
    M = Nb * HW

    s_h = _fold_scale(cv1_bn_gamma, cv1_bn_var)
    wh = (_w1x1(cv1_conv_w) * s_h[None, :]).astype(jnp.bfloat16)
    bh = (s_h * (cv1_conv_b - cv1_bn_mean) + cv1_bn_beta).reshape(1, -1)
    c_ = wh.shape[1]

    blocks = [
        (m0_cv1_conv_w, m0_cv1_conv_b, m0_cv1_bn_gamma, m0_cv1_bn_beta,
         m0_cv1_bn_mean, m0_cv1_bn_var, m0_cv2_conv_w, m0_cv2_conv_b,
         m0_cv2_bn_gamma, m0_cv2_bn_beta, m0_cv2_bn_mean, m0_cv2_bn_var),
        (m1_cv1_conv_w, m1_cv1_conv_b, m1_cv1_bn_gamma, m1_cv1_bn_beta,
         m1_cv1_bn_mean, m1_cv1_bn_var, m1_cv2_conv_w, m1_cv2_conv_b,
         m1_cv2_bn_gamma, m1_cv2_bn_beta, m1_cv2_bn_mean, m1_cv2_bn_var),
        (m2_cv1_conv_w, m2_cv1_conv_b, m2_cv1_bn_gamma, m2_cv1_bn_beta,
         m2_cv1_bn_mean, m2_cv1_bn_var, m2_cv2_conv_w, m2_cv2_conv_b,
         m2_cv2_bn_gamma, m2_cv2_bn_beta, m2_cv2_bn_mean, m2_cv2_bn_var),
    ]
    w1s, b1s, w2s, b2s = [], [], [], []
    for (w1, b1, g1, be1, mu1, v1, w2, b2, g2, be2, mu2, v2) in blocks:
        s1 = _fold_scale(g1, v1)
        w1s.append((_w1x1(w1) * s1[None, :]).astype(jnp.bfloat16))
        b1s.append((s1 * (b1 - mu1) + be1).reshape(1, -1))
        s2 = _fold_scale(g2, v2)
        taps = jnp.transpose(w2, (2, 3, 1, 0)).reshape(9, w2.shape[1], w2.shape[0])
        w2s.append((taps * s2[None, None, :]).astype(jnp.bfloat16))
        b2s.append((s2 * (b2 - mu2) + be2).reshape(1, -1))
    n_blocks = len(blocks)
    w1s = jnp.stack(w1s)
    b1s = jnp.stack(b1s)
    w2s = jnp.concatenate(
        [w.reshape(3, 3 * w.shape[1], w.shape[2]) for w in w2s], axis=0)
    b2s = jnp.stack(b2s)

    s_bn = _fold_scale(bn_gamma, bn_var)
    b_bn = bn_beta - bn_mean * s_bn
    w3 = _w1x1(cv3_w) * s_bn[None, :c_]
    w2o = _w1x1(cv2_w) * s_bn[None, c_:]
    wz = jnp.zeros((c_ + w2o.shape[0], 2 * c_), jnp.float32)
    wz = wz.at[:c_, :c_].set(w3).at[c_:, c_:].set(w2o).astype(jnp.bfloat16)
    bz = b_bn.reshape(1, -1)
    s4 = _fold_scale(cv4_bn_gamma, cv4_bn_var)
    w4 = (_w1x1(cv4_conv_w) * s4[None, :]).astype(jnp.bfloat16)
    b4 = (s4 * (cv4_conv_b - cv4_bn_mean) + cv4_bn_beta).reshape(1, -1)
    c2 = w4.shape[1]

    x2d = jnp.transpose(x, (0, 2, 3, 1)).reshape(M, c1)

    n_img = 4 if Nb % 4 == 0 else (2 if Nb % 2 == 0 else 1)
    kern = functools.partial(_csp_kernel, H=H, W=W, n_blocks=n_blocks,
                             n_img=n_img)
    rep = lambda i: (0, 0)
    rep3 = lambda i: (0, 0, 0)
    out = pl.pallas_call(
        kern,
        out_shape=jax.ShapeDtypeStruct((M, c2), jnp.float32),
        grid_spec=pltpu.PrefetchScalarGridSpec(
            num_scalar_prefetch=0,
            grid=(Nb // n_img,),
            in_specs=[
                pl.BlockSpec((n_img * HW, c1), lambda i: (i, 0)),
                pl.BlockSpec(wh.shape, rep), pl.BlockSpec(bh.shape, rep),
                pl.BlockSpec(w1s.shape, rep3), pl.BlockSpec(b1s.shape, rep3),
                pl.BlockSpec(w2s.shape, rep3), pl.BlockSpec(b2s.shape, rep3),
                pl.BlockSpec(wz.shape, rep), pl.BlockSpec(bz.shape, rep),
                pl.BlockSpec(w4.shape, rep), pl.BlockSpec(b4.shape, rep),
            ],
            out_specs=pl.BlockSpec((n_img * HW, c2), lambda i: (i, 0)),
            scratch_shapes=[pltpu.VMEM((HW + 2 * W, 3 * c_), jnp.bfloat16)
                            for _ in range(n_img)],
        ),
        compiler_params=pltpu.CompilerParams(
            dimension_semantics=("parallel",), vmem_limit_bytes=_VMEM_LIMIT),
    )(x2d, wh, bh, w1s, b1s, w2s, b2s, wz, bz, w4, b4)

    return jnp.transpose(out.reshape(Nb, H, W, c2), (0, 3, 1, 2))


# speedup vs baseline: 1.2958x; 1.2958x over previous
import functools

import jax
import jax.numpy as jnp
from jax.experimental import pallas as pl
from jax.experimental.pallas import tpu as pltpu

_NEG_SLOPE = 0.1
_BN_EPS = 1e-5
_VMEM_LIMIT = 60 * 1024 * 1024


def _leaky(v):
    return jnp.maximum(v, _NEG_SLOPE * v)


def _csp_kernel(x_ref, wh_ref, bh_ref, w1s_ref, b1s_ref, w2s_ref, b2s_ref,
                wz_ref, bz_ref, w4_ref, b4_ref,
                o_ref, *c3_refs, H, W, n_blocks, n_img):
    HW = H * W
    c_ = wh_ref.shape[1]
    col = jax.lax.broadcasted_iota(jnp.int32, (HW, 1), 0) % W
    m_left = (col != 0).astype(jnp.bfloat16)
    m_right = (col != (W - 1)).astype(jnp.bfloat16)

    scratches = c3_refs
    for img in range(n_img):
        c3_ref = scratches[img]
        base = img * HW
        xb = x_ref[base:base + HW, :].astype(jnp.bfloat16)

        y = _leaky((jnp.dot(xb, wh_ref[...],
                            preferred_element_type=jnp.float32)
                    + bh_ref[...]).astype(jnp.bfloat16))

        c3_ref[0:W + 1, :] = jnp.zeros((W + 1, 3 * c_), jnp.bfloat16)
        c3_ref[W + HW - 1:, :] = jnp.zeros((W + 1, 3 * c_), jnp.bfloat16)

        for blk in range(n_blocks):
            tb = _leaky((jnp.dot(y, w1s_ref[blk],
                                 preferred_element_type=jnp.float32)
                         + b1s_ref[blk]).astype(jnp.bfloat16))
            tl = tb * m_right
            tr = tb * m_left
            c3_ref[W + 1:W + 1 + HW, 0:c_] = tl
            c3_ref[W:W + HW, c_:2 * c_] = tb
            c3_ref[W - 1:W - 1 + HW, 2 * c_:3 * c_] = tr
            acc = jnp.dot(c3_ref[0:HW, :], w2s_ref[3 * blk],
                          preferred_element_type=jnp.float32)
            acc = acc + jnp.dot(c3_ref[W:W + HW, :], w2s_ref[3 * blk + 1],
                                preferred_element_type=jnp.float32)
            acc = acc + jnp.dot(c3_ref[2 * W:2 * W + HW, :],
                                w2s_ref[3 * blk + 2],
                                preferred_element_type=jnp.float32)
            y = _leaky((acc + b2s_ref[blk]).astype(jnp.bfloat16)) + y

        zin = jnp.concatenate([y, xb], axis=1)
        u = _leaky((jnp.dot(zin, wz_ref[...],
                            preferred_element_type=jnp.float32)
                    + bz_ref[...]).astype(jnp.bfloat16))
        v = (jnp.dot(u, w4_ref[...], preferred_element_type=jnp.float32)
             + b4_ref[...])
        o_ref[base:base + HW, :] = _leaky(v)


def _w1x1(w):
    return jnp.transpose(w[:, :, 0, 0], (1, 0))


def _fold_scale(gamma, var):
    return gamma * jax.lax.rsqrt(var + _BN_EPS)


def kernel(x, cv1_conv_w, cv1_conv_b, cv1_bn_gamma, cv1_bn_beta, cv1_bn_mean, cv1_bn_var, cv2_w, cv3_w, cv4_conv_w, cv4_conv_b, cv4_bn_gamma, cv4_bn_beta, cv4_bn_mean, cv4_bn_var, bn_gamma, bn_beta, bn_mean, bn_var, m0_cv1_conv_w, m0_cv1_conv_b, m0_cv1_bn_gamma, m0_cv1_bn_beta, m0_cv1_bn_mean, m0_cv1_bn_var, m0_cv2_conv_w, m0_cv2_conv_b, m0_cv2_bn_gamma, m0_cv2_bn_beta, m0_cv2_bn_mean, m0_cv2_bn_var, m1_cv1_conv_w, m1_cv1_conv_b, m1_cv1_bn_gamma, m1_cv1_bn_beta, m1_cv1_bn_mean, m1_cv1_bn_var, m1_cv2_conv_w, m1_cv2_conv_b, m1_cv2_bn_gamma, m1_cv2_bn_beta, m1_cv2_bn_mean, m1_cv2_bn_var, m2_cv1_conv_w, m2_cv1_conv_b, m2_cv1_bn_gamma, m2_cv1_bn_beta, m2_cv1_bn_mean, m2_cv1_bn_var, m2_cv2_conv_w, m2_cv2_conv_b, m2_cv2_bn_gamma, m2_cv2_bn_beta, m2_cv2_bn_mean, m2_cv2_bn_var):
    Nb, c1, H, W = x.shape
    HW = H * W
    M = Nb * HW

    s_h = _fold_scale(cv1_bn_gamma, cv1_bn_var)
    wh = (_w1x1(cv1_conv_w) * s_h[None, :]).astype(jnp.bfloat16)
    bh = (s_h * (cv1_conv_b - cv1_bn_mean) + cv1_bn_beta).reshape(1, -1)
    c_ = wh.shape[1]

    blocks = [
        (m0_cv1_conv_w, m0_cv1_conv_b, m0_cv1_bn_gamma, m0_cv1_bn_beta,
         m0_cv1_bn_mean, m0_cv1_bn_var, m0_cv2_conv_w, m0_cv2_conv_b,
         m0_cv2_bn_gamma, m0_cv2_bn_beta, m0_cv2_bn_mean, m0_cv2_bn_var),
        (m1_cv1_conv_w, m1_cv1_conv_b, m1_cv1_bn_gamma, m1_cv1_bn_beta,
         m1_cv1_bn_mean, m1_cv1_bn_var, m1_cv2_conv_w, m1_cv2_conv_b,
         m1_cv2_bn_gamma, m1_cv2_bn_beta, m1_cv2_bn_mean, m1_cv2_bn_var),
        (m2_cv1_conv_w, m2_cv1_conv_b, m2_cv1_bn_gamma, m2_cv1_bn_beta,
         m2_cv1_bn_mean, m2_cv1_bn_var, m2_cv2_conv_w, m2_cv2_conv_b,
         m2_cv2_bn_gamma, m2_cv2_bn_beta, m2_cv2_bn_mean, m2_cv2_bn_var),
    ]
    w1s, b1s, w2s, b2s = [], [], [], []
    for (w1, b1, g1, be1, mu1, v1, w2, b2, g2, be2, mu2, v2) in blocks:
        s1 = _fold_scale(g1, v1)
        w1s.append((_w1x1(w1) * s1[None, :]).astype(jnp.bfloat16))
        b1s.append((s1 * (b1 - mu1) + be1).reshape(1, -1))
        s2 = _fold_scale(g2, v2)
        taps = jnp.transpose(w2, (2, 3, 1, 0)).reshape(9, w2.shape[1], w2.shape[0])
        w2s.append((taps * s2[None, None, :]).astype(jnp.bfloat16))
        b2s.append((s2 * (b2 - mu2) + be2).reshape(1, -1))
    n_blocks = len(blocks)
    w1s = jnp.stack(w1s)
    b1s = jnp.stack(b1s)
    w2s = jnp.concatenate(
        [w.reshape(3, 3 * w.shape[1], w.shape[2]) for w in w2s], axis=0)
    b2s = jnp.stack(b2s)

    s_bn = _fold_scale(bn_gamma, bn_var)
    b_bn = bn_beta - bn_mean * s_bn
    w3 = _w1x1(cv3_w) * s_bn[None, :c_]
    w2o = _w1x1(cv2_w) * s_bn[None, c_:]
    wz = jnp.zeros((c_ + w2o.shape[0], 2 * c_), jnp.float32)
    wz = wz.at[:c_, :c_].set(w3).at[c_:, c_:].set(w2o).astype(jnp.bfloat16)
    bz = b_bn.reshape(1, -1)
    s4 = _fold_scale(cv4_bn_gamma, cv4_bn_var)
    w4 = (_w1x1(cv4_conv_w) * s4[None, :]).astype(jnp.bfloat16)
    b4 = (s4 * (cv4_conv_b - cv4_bn_mean) + cv4_bn_beta).reshape(1, -1)
    c2 = w4.shape[1]

    x2d = jnp.transpose(x, (0, 2, 3, 1)).reshape(M, c1)

    n_img = 2 if Nb % 2 == 0 else 1
    kern = functools.partial(_csp_kernel, H=H, W=W, n_blocks=n_blocks,
                             n_img=n_img)
    rep = lambda i: (0, 0)
    rep3 = lambda i: (0, 0, 0)
    out = pl.pallas_call(
        kern,
        out_shape=jax.ShapeDtypeStruct((M, c2), jnp.float32),
        grid_spec=pltpu.PrefetchScalarGridSpec(
            num_scalar_prefetch=0,
            grid=(Nb // n_img,),
            in_specs=[
                pl.BlockSpec((n_img * HW, c1), lambda i: (i, 0)),
                pl.BlockSpec(wh.shape, rep), pl.BlockSpec(bh.shape, rep),
                pl.BlockSpec(w1s.shape, rep3), pl.BlockSpec(b1s.shape, rep3),
                pl.BlockSpec(w2s.shape, rep3), pl.BlockSpec(b2s.shape, rep3),
                pl.BlockSpec(wz.shape, rep), pl.BlockSpec(bz.shape, rep),
                pl.BlockSpec(w4.shape, rep), pl.BlockSpec(b4.shape, rep),
            ],
            out_specs=pl.BlockSpec((n_img * HW, c2), lambda i: (i, 0)),
            scratch_shapes=[pltpu.VMEM((HW + 2 * W, 3 * c_), jnp.bfloat16)
                            for _ in range(n_img)],
        ),
        compiler_params=pltpu.CompilerParams(
            dimension_semantics=("parallel",), vmem_limit_bytes=_VMEM_LIMIT),
    )(x2d, wh, bh, w1s, b1s, w2s, b2s, wz, bz, w4, b4)

    return jnp.transpose(out.reshape(Nb, H, W, c2), (0, 3, 1, 2))


# speedup vs baseline: 1.3273x; 1.0244x over previous
import functools

import jax
import jax.numpy as jnp
from jax.experimental import pallas as pl
from jax.experimental.pallas import tpu as pltpu

_NEG_SLOPE = 0.1
_BN_EPS = 1e-5
_VMEM_LIMIT = 48 * 1024 * 1024


def _leaky(v):
    return jnp.maximum(v, _NEG_SLOPE * v)


def _csp_kernel(x_ref, wh_ref, bh_ref, w1s_ref, b1s_ref, w2s_ref, b2s_ref,
                wz_ref, bz_ref, w4_ref, b4_ref,
                o_ref, *c3_refs, H, W, n_blocks, n_img):
    HW = H * W
    c_ = wh_ref.shape[1]
    col = jax.lax.broadcasted_iota(jnp.int32, (HW, 1), 0) % W
    m_left = (col != 0).astype(jnp.bfloat16)
    m_right = (col != (W - 1)).astype(jnp.bfloat16)

    scratches = c3_refs
    for img in range(n_img):
        c3_ref = scratches[img]
        base = img * HW
        xb = x_ref[base:base + HW, :].astype(jnp.bfloat16)

        y = _leaky((jnp.dot(xb, wh_ref[...],
                            preferred_element_type=jnp.float32)
                    + bh_ref[...]).astype(jnp.bfloat16))

        c3_ref[0:W + 1, :] = jnp.zeros((W + 1, 3 * c_), jnp.bfloat16)
        c3_ref[W + HW - 1:, :] = jnp.zeros((W + 1, 3 * c_), jnp.bfloat16)

        for blk in range(n_blocks):
            tb = _leaky((jnp.dot(y, w1s_ref[blk],
                                 preferred_element_type=jnp.float32)
                         + b1s_ref[blk]).astype(jnp.bfloat16))
            tl = tb * m_right
            tr = tb * m_left
            c3_ref[W + 1:W + 1 + HW, 0:c_] = tl
            c3_ref[W:W + HW, c_:2 * c_] = tb
            c3_ref[W - 1:W - 1 + HW, 2 * c_:3 * c_] = tr
            acc = jnp.dot(c3_ref[0:HW, :], w2s_ref[3 * blk],
                          preferred_element_type=jnp.float32)
            acc = acc + jnp.dot(c3_ref[W:W + HW, :], w2s_ref[3 * blk + 1],
                                preferred_element_type=jnp.float32)
            acc = acc + jnp.dot(c3_ref[2 * W:2 * W + HW, :],
                                w2s_ref[3 * blk + 2],
                                preferred_element_type=jnp.float32)
            y = _leaky((acc + b2s_ref[blk]).astype(jnp.bfloat16)) + y

        zin = jnp.concatenate([y, xb], axis=1)
        u = _leaky((jnp.dot(zin, wz_ref[...],
                            preferred_element_type=jnp.float32)
                    + bz_ref[...]).astype(jnp.bfloat16))
        v = (jnp.dot(u, w4_ref[...], preferred_element_type=jnp.float32)
             + b4_ref[...])
        o_ref[base:base + HW, :] = _leaky(v)


def _w1x1(w):
    return jnp.transpose(w[:, :, 0, 0], (1, 0))


def _fold_scale(gamma, var):
    return gamma * jax.lax.rsqrt(var + _BN_EPS)


def kernel(x, cv1_conv_w, cv1_conv_b, cv1_bn_gamma, cv1_bn_beta, cv1_bn_mean, cv1_bn_var, cv2_w, cv3_w, cv4_conv_w, cv4_conv_b, cv4_bn_gamma, cv4_bn_beta, cv4_bn_mean, cv4_bn_var, bn_gamma, bn_beta, bn_mean, bn_var, m0_cv1_conv_w, m0_cv1_conv_b, m0_cv1_bn_gamma, m0_cv1_bn_beta, m0_cv1_bn_mean, m0_cv1_bn_var, m0_cv2_conv_w, m0_cv2_conv_b, m0_cv2_bn_gamma, m0_cv2_bn_beta, m0_cv2_bn_mean, m0_cv2_bn_var, m1_cv1_conv_w, m1_cv1_conv_b, m1_cv1_bn_gamma, m1_cv1_bn_beta, m1_cv1_bn_mean, m1_cv1_bn_var, m1_cv2_conv_w, m1_cv2_conv_b, m1_cv2_bn_gamma, m1_cv2_bn_beta, m1_cv2_bn_mean, m1_cv2_bn_var, m2_cv1_conv_w, m2_cv1_conv_b, m2_cv1_bn_gamma, m2_cv1_bn_beta, m2_cv1_bn_mean, m2_cv1_bn_var, m2_cv2_conv_w, m2_cv2_conv_b, m2_cv2_bn_gamma, m2_cv2_bn_beta, m2_cv2_bn_mean, m2_cv2_bn_var):
    Nb, c1, H, W = x.shape
    HW = H * W
    M = Nb * HW

    s_h = _fold_scale(cv1_bn_gamma, cv1_bn_var)
    wh = (_w1x1(cv1_conv_w) * s_h[None, :]).astype(jnp.bfloat16)
    bh = (s_h * (cv1_conv_b - cv1_bn_mean) + cv1_bn_beta).reshape(1, -1)
    c_ = wh.shape[1]

    blocks = [
        (m0_cv1_conv_w, m0_cv1_conv_b, m0_cv1_bn_gamma, m0_cv1_bn_beta,
         m0_cv1_bn_mean, m0_cv1_bn_var, m0_cv2_conv_w, m0_cv2_conv_b,
         m0_cv2_bn_gamma, m0_cv2_bn_beta, m0_cv2_bn_mean, m0_cv2_bn_var),
        (m1_cv1_conv_w, m1_cv1_conv_b, m1_cv1_bn_gamma, m1_cv1_bn_beta,
         m1_cv1_bn_mean, m1_cv1_bn_var, m1_cv2_conv_w, m1_cv2_conv_b,
         m1_cv2_bn_gamma, m1_cv2_bn_beta, m1_cv2_bn_mean, m1_cv2_bn_var),
        (m2_cv1_conv_w, m2_cv1_conv_b, m2_cv1_bn_gamma, m2_cv1_bn_beta,
         m2_cv1_bn_mean, m2_cv1_bn_var, m2_cv2_conv_w, m2_cv2_conv_b,
         m2_cv2_bn_gamma, m2_cv2_bn_beta, m2_cv2_bn_mean, m2_cv2_bn_var),
    ]
    w1s, b1s, w2s, b2s = [], [], [], []
    for (w1, b1, g1, be1, mu1, v1, w2, b2, g2, be2, mu2, v2) in blocks:
        s1 = _fold_scale(g1, v1)
        w1s.append((_w1x1(w1) * s1[None, :]).astype(jnp.bfloat16))
        b1s.append((s1 * (b1 - mu1) + be1).reshape(1, -1))
        s2 = _fold_scale(g2, v2)
        taps = jnp.transpose(w2, (2, 3, 1, 0)).reshape(9, w2.shape[1], w2.shape[0])
        w2s.append((taps * s2[None, None, :]).astype(jnp.bfloat16))
        b2s.append((s2 * (b2 - mu2) + be2).reshape(1, -1))
    n_blocks = len(blocks)
    w1s = jnp.stack(w1s)
    b1s = jnp.stack(b1s)
    w2s = jnp.concatenate(
        [w.reshape(3, 3 * w.shape[1], w.shape[2]) for w in w2s], axis=0)
    b2s = jnp.stack(b2s)

    s_bn = _fold_scale(bn_gamma, bn_var)
    b_bn = bn_beta - bn_mean * s_bn
    w3 = _w1x1(cv3_w) * s_bn[None, :c_]
    w2o = _w1x1(cv2_w) * s_bn[None, c_:]
    wz = jnp.zeros((c_ + w2o.shape[0], 2 * c_), jnp.float32)
    wz = wz.at[:c_, :c_].set(w3).at[c_:, c_:].set(w2o).astype(jnp.bfloat16)
    bz = b_bn.reshape(1, -1)
    s4 = _fold_scale(cv4_bn_gamma, cv4_bn_var)
    w4 = (_w1x1(cv4_conv_w) * s4[None, :]).astype(jnp.bfloat16)
    b4 = (s4 * (cv4_conv_b - cv4_bn_mean) + cv4_bn_beta).reshape(1, -1)
    c2 = w4.shape[1]

    x2d = jnp.transpose(x, (0, 2, 3, 1)).reshape(M, c1)

    n_img = 2 if Nb % 2 == 0 else 1
    kern = functools.partial(_csp_kernel, H=H, W=W, n_blocks=n_blocks,
                             n_img=n_img)
    rep = lambda i: (0, 0)
    rep3 = lambda i: (0, 0, 0)
    out = pl.pallas_call(
        kern,
        out_shape=jax.ShapeDtypeStruct((M, c2), jnp.float32),
        grid_spec=pltpu.PrefetchScalarGridSpec(
            num_scalar_prefetch=0,
            grid=(Nb // n_img,),
            in_specs=[
                pl.BlockSpec((n_img * HW, c1), lambda i: (i, 0)),
                pl.BlockSpec(wh.shape, rep), pl.BlockSpec(bh.shape, rep),
                pl.BlockSpec(w1s.shape, rep3), pl.BlockSpec(b1s.shape, rep3),
                pl.BlockSpec(w2s.shape, rep3), pl.BlockSpec(b2s.shape, rep3),
                pl.BlockSpec(wz.shape, rep), pl.BlockSpec(bz.shape, rep),
                pl.BlockSpec(w4.shape, rep), pl.BlockSpec(b4.shape, rep),
            ],
            out_specs=pl.BlockSpec((n_img * HW, c2), lambda i: (i, 0)),
            scratch_shapes=[pltpu.VMEM((HW + 2 * W, 3 * c_), jnp.bfloat16)
                            for _ in range(n_img)],
        ),
        compiler_params=pltpu.CompilerParams(
            dimension_semantics=("parallel",), vmem_limit_bytes=_VMEM_LIMIT),
    )(x2d, wh, bh, w1s, b1s, w2s, b2s, wz, bz, w4, b4)

    return jnp.transpose(out.reshape(Nb, H, W, c2), (0, 3, 1, 2))
